# baseline (speedup 1.0000x reference)
import jax
import jax.numpy as jnp
from jax.experimental import pallas as pl

B, T, N, FIN, FOUT = 4, 8, 512, 128, 64
BT = B * T


BT_PER = 32


def _gat_body(inp_ref, mask_ref, w_ref, a1_ref, a2_ref, out_ref):
    wb = w_ref[...].astype(jnp.bfloat16)
    mask = mask_ref[...]
    a1 = a1_ref[...]
    a2 = a2_ref[...]
    for k in range(BT_PER):
        xb = inp_ref[k].astype(jnp.bfloat16)
        ht = jax.lax.dot_general(wb, xb, (((0,), (1,)), ((), ())),
                                 preferred_element_type=jnp.float32)
        h2t = jnp.dot(ht.astype(jnp.bfloat16), mask,
                      preferred_element_type=jnp.float32)
        score = (jnp.sum(ht * a1, axis=0)
                 + jnp.sum(h2t * a2, axis=0))
        colsum = jnp.sum(ht, axis=1)
        out_ref[k] = jnp.maximum(score[:, None] * colsum[None, :], 0.0)


def kernel(inp, adj, W, a):
    f = W.shape[1]
    inp_r = inp.reshape(BT, N, FIN)
    mask_b = (adj > 0).astype(jnp.bfloat16)
    a1 = a[:f, :]
    a2 = a[f:, :]

    out = pl.pallas_call(
        _gat_body,
        grid=(BT // BT_PER,),
        in_specs=[
            pl.BlockSpec((BT_PER, N, FIN), lambda i: (i, 0, 0)),
            pl.BlockSpec((N, N), lambda i: (0, 0)),
            pl.BlockSpec((FIN, f), lambda i: (0, 0)),
            pl.BlockSpec((f, N), lambda i: (0, 0)),
            pl.BlockSpec((f, N), lambda i: (0, 0)),
        ],
        out_specs=pl.BlockSpec((BT_PER, N, f), lambda i: (i, 0, 0)),
        out_shape=jax.ShapeDtypeStruct((BT, N, f), jnp.float32),
    )(inp_r, mask_b, W, a1, a2)

    return out.reshape(B, T, N, f)


# speedup vs baseline: 1.0516x; 1.0516x over previous
import jax
import jax.numpy as jnp
from jax.experimental import pallas as pl
from jax.experimental.pallas import tpu as pltpu

B, T, N, FIN, FOUT = 4, 8, 512, 128, 64
BT = B * T
C = 4
NCH = BT // C


def _gat_body(inp_hbm, mask_ref, w_ref, a1_ref, a2_ref, out_hbm,
              xbuf, obuf, in_sem, out_sem):
    def in_copy(c, slot):
        return pltpu.make_async_copy(
            inp_hbm.at[pl.ds(c * C, C)], xbuf.at[slot], in_sem.at[slot])

    def out_copy(c, slot):
        return pltpu.make_async_copy(
            obuf.at[slot], out_hbm.at[pl.ds(c * C, C)], out_sem.at[slot])

    in_copy(0, 0).start()
    in_copy(1, 1).start()
    wb = w_ref[...].astype(jnp.bfloat16)
    mask = mask_ref[...]
    a1 = a1_ref[...]
    a2 = a2_ref[...]
    for c in range(NCH):
        slot = c % 2
        in_copy(c, slot).wait()
        if c >= 2:
            out_copy(c - 2, slot).wait()
        for k in range(C):
            xb = xbuf[slot, k].astype(jnp.bfloat16)
            ht = jax.lax.dot_general(wb, xb, (((0,), (1,)), ((), ())),
                                     preferred_element_type=jnp.float32)
            h2t = jnp.dot(ht.astype(jnp.bfloat16), mask,
                          preferred_element_type=jnp.float32)
            score = (jnp.sum(ht * a1, axis=0)
                     + jnp.sum(h2t * a2, axis=0))
            colsum = jnp.sum(ht, axis=1)
            obuf[slot, k] = jnp.maximum(score[:, None] * colsum[None, :], 0.0)
        out_copy(c, slot).start()
        if c + 2 < NCH:
            in_copy(c + 2, slot).start()
    out_copy(NCH - 2, 0).wait()
    out_copy(NCH - 1, 1).wait()


def kernel(inp, adj, W, a):
    f = W.shape[1]
    inp_r = inp.reshape(BT, N, FIN)
    mask_b = (adj > 0).astype(jnp.bfloat16)
    a1 = a[:f, :]
    a2 = a[f:, :]

    out = pl.pallas_call(
        _gat_body,
        in_specs=[
            pl.BlockSpec(memory_space=pl.ANY),
            pl.BlockSpec(memory_space=pltpu.MemorySpace.VMEM),
            pl.BlockSpec(memory_space=pltpu.MemorySpace.VMEM),
            pl.BlockSpec(memory_space=pltpu.MemorySpace.VMEM),
            pl.BlockSpec(memory_space=pltpu.MemorySpace.VMEM),
        ],
        out_specs=pl.BlockSpec(memory_space=pl.ANY),
        out_shape=jax.ShapeDtypeStruct((BT, N, f), jnp.float32),
        scratch_shapes=[
            pltpu.VMEM((2, C, N, FIN), jnp.float32),
            pltpu.VMEM((2, C, N, FOUT), jnp.float32),
            pltpu.SemaphoreType.DMA((2,)),
            pltpu.SemaphoreType.DMA((2,)),
        ],
    )(inp_r, mask_b, W, a1, a2)

    return out.reshape(B, T, N, f)


# speedup vs baseline: 1.2342x; 1.1736x over previous
import jax
import jax.numpy as jnp
from jax.experimental import pallas as pl
from jax.experimental.pallas import tpu as pltpu

B, T, N, FIN, FOUT = 4, 8, 512, 128, 64
BT = B * T
C = 4
NCH = BT // C
NSLOT = 3


def _gat_body(inp_hbm, adj_ref, w_ref, a_ref, out_hbm,
              xbuf, obuf, in_sem, out_sem):
    def in_copy(c):
        slot = c % NSLOT
        return pltpu.make_async_copy(
            inp_hbm.at[pl.ds(c * C, C)], xbuf.at[slot], in_sem.at[slot])

    def out_copy(c):
        slot = c % NSLOT
        return pltpu.make_async_copy(
            obuf.at[slot], out_hbm.at[pl.ds(c * C, C)], out_sem.at[slot])

    for c in range(NSLOT):
        in_copy(c).start()

    mask = (adj_ref[...] > 0).astype(jnp.bfloat16)
    wb = w_ref[...].astype(jnp.bfloat16)
    a1 = a_ref[:FOUT, :]
    a2 = a_ref[FOUT:, :]

    for c in range(NCH):
        slot = c % NSLOT
        in_copy(c).wait()
        if c >= NSLOT:
            out_copy(c - NSLOT).wait()
        for k in range(C):
            xb = xbuf[slot, k].astype(jnp.bfloat16)
            ht = jax.lax.dot_general(wb, xb, (((0,), (1,)), ((), ())),
                                     preferred_element_type=jnp.float32)
            h2t = jnp.dot(ht.astype(jnp.bfloat16), mask,
                          preferred_element_type=jnp.float32)
            score = (jnp.sum(ht * a1, axis=0)
                     + jnp.sum(h2t * a2, axis=0))
            colsum = jnp.sum(ht, axis=1)
            obuf[slot, k] = jnp.maximum(score[:, None] * colsum[None, :], 0.0)
        out_copy(c).start()
        if c + NSLOT < NCH:
            in_copy(c + NSLOT).start()

    for c in range(NCH - NSLOT, NCH):
        out_copy(c).wait()


def kernel(inp, adj, W, a):
    f = W.shape[1]
    inp_r = inp.reshape(BT, N, FIN)

    out = pl.pallas_call(
        _gat_body,
        in_specs=[
            pl.BlockSpec(memory_space=pl.ANY),
            pl.BlockSpec(memory_space=pltpu.MemorySpace.VMEM),
            pl.BlockSpec(memory_space=pltpu.MemorySpace.VMEM),
            pl.BlockSpec(memory_space=pltpu.MemorySpace.VMEM),
        ],
        out_specs=pl.BlockSpec(memory_space=pl.ANY),
        out_shape=jax.ShapeDtypeStruct((BT, N, f), jnp.float32),
        scratch_shapes=[
            pltpu.VMEM((NSLOT, C, N, FIN), jnp.float32),
            pltpu.VMEM((NSLOT, C, N, FOUT), jnp.float32),
            pltpu.SemaphoreType.DMA((NSLOT,)),
            pltpu.SemaphoreType.DMA((NSLOT,)),
        ],
    )(inp_r, adj, W, a)

    return out.reshape(B, T, N, f)
